# baseline (speedup 1.0000x reference)
import jax
import jax.numpy as jnp
from jax.experimental import pallas as pl
from jax.experimental.pallas import tpu as pltpu

N_PROXIES = 8192
N_CAMS = 8
PPC = N_PROXIES // N_CAMS
TEMP = 0.07
B = 1024
D = 256
TILE = 128
P = 2048
NT = P // TILE


def _tile_kernel(scal_ref, feat_ref, mem_ref, cam_ref, tgt_ref, out_ref):
    feat = feat_ref[...]
    camv = cam_ref[...]
    camsub = jax.lax.broadcasted_iota(jnp.int32, (N_CAMS, B), 0)
    ohi = (camsub == camv).astype(jnp.float32)
    cnt = jnp.sum(ohi, axis=1, keepdims=True)
    padded = jnp.floor((cnt + (TILE - 1)) * (1.0 / TILE)) * TILE
    r8 = jax.lax.broadcasted_iota(jnp.int32, (N_CAMS, N_CAMS), 0)
    c8 = jax.lax.broadcasted_iota(jnp.int32, (N_CAMS, N_CAMS), 1)
    strict_lt = (c8 < r8).astype(jnp.float32)
    starts = jax.lax.dot_general(
        strict_lt, padded, (((1,), (0,)), ((), ())),
        preferred_element_type=jnp.float32)
    ri = jax.lax.broadcasted_iota(jnp.int32, (B, B), 0)
    ci = jax.lax.broadcasted_iota(jnp.int32, (B, B), 1)
    lt_inc = (ri <= ci).astype(jnp.float32)
    incl = jax.lax.dot_general(
        ohi, lt_inc, (((1,), (0,)), ((), ())),
        preferred_element_type=jnp.float32)
    rank = jnp.sum(ohi * (incl - 1.0), axis=0, keepdims=True)
    pos = jnp.sum(ohi * starts, axis=0, keepdims=True) + rank
    winv = jnp.sum(jnp.where(ohi > 0, 1.0 / cnt, 0.0), axis=0,
                   keepdims=True)
    tgtf = tgt_ref[...].astype(jnp.float32)
    lane_b = jax.lax.broadcasted_iota(jnp.int32, (B, 128), 1)
    tgt_col = jnp.transpose(tgtf)
    tw_tab = (jnp.where(lane_b == 0, jnp.floor(tgt_col * 0.25), 0.0)
              + jnp.where(lane_b == 1, tgt_col - 4.0 * jnp.floor(tgt_col * 0.25), 0.0)
              + jnp.where(lane_b == 2, jnp.transpose(winv), 0.0))

    def body(t, acc):
        c = scal_ref[t]
        pj = (jax.lax.broadcasted_iota(jnp.int32, (TILE, 1), 0)
              + t * TILE).astype(jnp.float32)
        gb = pos == pj
        g = gb.astype(jnp.float32)
        x = jax.lax.dot_general(
            g, feat, (((1,), (0,)), ((), ())),
            preferred_element_type=jnp.float32)
        tw = jax.lax.dot_general(
            g, tw_tab, (((1,), (0,)), ((), ())),
            preferred_element_type=jnp.float32)
        tgt_t = 4.0 * tw[:, 0:1] + tw[:, 1:2] + 0.5
        w_t = tw[:, 2:3]
        w = mem_ref[pl.ds(c * PPC, PPC), :]
        sim = jax.lax.dot_general(
            x, w, (((1,), (1,)), ((), ())), preferred_element_type=jnp.float32
        ) * (1.0 / TEMP)
        lse = jnp.log(jnp.sum(jnp.exp(sim), axis=1, keepdims=True))
        cols = jax.lax.broadcasted_iota(jnp.int32, (TILE, PPC), 1)
        tlogit = jnp.sum(jnp.where(cols == tgt_t.astype(jnp.int32), sim, 0.0),
                         axis=1, keepdims=True)
        return acc + (lse - tlogit) * w_t

    n_real = scal_ref[NT]
    acc = jax.lax.fori_loop(0, n_real, body, jnp.zeros((TILE, 1), jnp.float32))
    lane = jax.lax.broadcasted_iota(jnp.int32, (1, 128), 1)
    out_ref[...] = jnp.where(lane == 0, jnp.sum(acc), 0.0)


def kernel(batch_feat, abs_proxy_label, camid, pseudo_cluster_label, memory,
           epoch, k, inter_loss_epoch):
    camid = camid.astype(jnp.int32)
    local_tgt = (abs_proxy_label % PPC).astype(jnp.int32)

    cams = jnp.arange(N_CAMS, dtype=jnp.int32)
    cnt = jnp.sum((camid[None, :] == cams[:, None]).astype(jnp.int32), axis=1)
    padded = ((cnt + TILE - 1) // TILE) * TILE
    ends = jnp.sum(jnp.where(cams[None, :] <= cams[:, None], padded[None, :], 0),
                   axis=1)
    tile_start = jnp.arange(NT, dtype=jnp.int32) * TILE
    tile_cam = jnp.minimum(
        jnp.sum((tile_start[:, None] >= ends[None, :]).astype(jnp.int32),
                axis=1), N_CAMS - 1)
    n_real = ends[N_CAMS - 1] // TILE
    scalars = jnp.concatenate([tile_cam, n_real[None]]).astype(jnp.int32)

    out = pl.pallas_call(
        _tile_kernel,
        grid_spec=pltpu.PrefetchScalarGridSpec(
            num_scalar_prefetch=1,
            grid=(1,),
            in_specs=[
                pl.BlockSpec((B, D), lambda i, tc: (0, 0)),
                pl.BlockSpec((N_PROXIES, D), lambda i, tc: (0, 0)),
                pl.BlockSpec((1, B), lambda i, tc: (0, 0)),
                pl.BlockSpec((1, B), lambda i, tc: (0, 0)),
            ],
            out_specs=pl.BlockSpec((1, 128), lambda i, tc: (0, 0)),
        ),
        out_shape=jax.ShapeDtypeStruct((1, 128), jnp.float32),
    )(scalars, batch_feat, memory,
      camid.reshape(1, B), local_tgt.reshape(1, B))
    return out[0, 0]


# speedup vs baseline: 1.0008x; 1.0008x over previous
import functools

import jax
import jax.numpy as jnp
from jax import lax
from jax.experimental import pallas as pl
from jax.experimental.pallas import tpu as pltpu
from jax.experimental.pallas import tpu_sc as plsc

N_PROXIES = 8192
N_CAMS = 8
PPC = N_PROXIES // N_CAMS
TEMP = 0.07
B = 1024
D = 256
TILE = 128
P = 2048
NT = P // TILE


def _sc_gather(table_hbm, idx_hbm, out_hbm, idx_v, rows_v, sem):
    nc = lax.axis_size("c")
    wid = lax.axis_index("s") * nc + lax.axis_index("c")
    bpw = idx_v.shape[0]
    base = wid * bpw
    pltpu.sync_copy(idx_hbm.at[pl.ds(base, bpw)], idx_v)
    pltpu.async_copy(table_hbm.at[idx_v], rows_v, sem).wait()
    pltpu.sync_copy(rows_v, out_hbm.at[pl.ds(base, bpw)])


def _target_rows(memory, abs_idx):
    info = plsc.get_sparse_core_info()
    nw = info.num_cores * info.num_subcores
    bpw = B // nw
    mesh = plsc.VectorSubcoreMesh(core_axis_name="c", subcore_axis_name="s")
    return functools.partial(
        pl.kernel, mesh=mesh,
        out_type=jax.ShapeDtypeStruct((B, D), jnp.float32),
        scratch_types=[
            pltpu.VMEM((bpw,), jnp.int32),
            pltpu.VMEM((bpw, D), jnp.float32),
            pltpu.SemaphoreType.DMA,
        ],
    )(_sc_gather)(memory, abs_idx)


def _tile_kernel(scal_ref, feat_ref, mem_ref, cam_ref, mabs_ref, out_ref):
    feat = feat_ref[...]
    camv = cam_ref[...]
    camsub = jax.lax.broadcasted_iota(jnp.int32, (N_CAMS, B), 0)
    ohi = (camsub == camv).astype(jnp.float32)
    cnt = jnp.sum(ohi, axis=1, keepdims=True)
    padded = jnp.floor((cnt + (TILE - 1)) * (1.0 / TILE)) * TILE
    r8 = jax.lax.broadcasted_iota(jnp.int32, (N_CAMS, N_CAMS), 0)
    c8 = jax.lax.broadcasted_iota(jnp.int32, (N_CAMS, N_CAMS), 1)
    strict_lt = (c8 < r8).astype(jnp.float32)
    starts = jax.lax.dot_general(
        strict_lt, padded, (((1,), (0,)), ((), ())),
        preferred_element_type=jnp.float32)
    ri = jax.lax.broadcasted_iota(jnp.int32, (B, B), 0)
    ci = jax.lax.broadcasted_iota(jnp.int32, (B, B), 1)
    lt_inc = (ri <= ci).astype(jnp.float32)
    incl = jax.lax.dot_general(
        ohi, lt_inc, (((1,), (0,)), ((), ())),
        preferred_element_type=jnp.float32)
    rank = jnp.sum(ohi * (incl - 1.0), axis=0, keepdims=True)
    pos = jnp.sum(ohi * starts, axis=0, keepdims=True) + rank
    winv = jnp.sum(jnp.where(ohi > 0, 1.0 / cnt, 0.0), axis=0,
                   keepdims=True)
    d = jnp.sum(feat * mabs_ref[...], axis=1, keepdims=True) * (1.0 / TEMP)
    tterm = jax.lax.dot_general(
        winv, d, (((1,), (0,)), ((), ())),
        preferred_element_type=jnp.float32)

    def body(t, acc):
        c = scal_ref[t]
        pj = (jax.lax.broadcasted_iota(jnp.int32, (TILE, 1), 0)
              + t * TILE).astype(jnp.float32)
        g = (pos == pj).astype(jnp.float32)
        x = jax.lax.dot_general(
            g, feat, (((1,), (0,)), ((), ())),
            preferred_element_type=jnp.float32)
        w = mem_ref[pl.ds(c * PPC, PPC), :]
        sim = jax.lax.dot_general(
            x, w, (((1,), (1,)), ((), ())), preferred_element_type=jnp.float32
        ) * (1.0 / TEMP)
        lse = jnp.log(jnp.sum(jnp.exp(sim), axis=1, keepdims=True))
        vb = scal_ref[NT + 1 + t]
        wc = 1.0 / lax.convert_element_type(scal_ref[2 * NT + 1 + t],
                                            jnp.float32)
        rj = jax.lax.broadcasted_iota(jnp.int32, (TILE, 1), 0)
        return acc + jnp.where(rj < vb, lse * wc, 0.0)

    n_real = scal_ref[NT]
    acc = jax.lax.fori_loop(0, n_real, body, jnp.zeros((TILE, 1), jnp.float32))
    lane = jax.lax.broadcasted_iota(jnp.int32, (1, 128), 1)
    out_ref[...] = jnp.where(lane == 0, jnp.sum(acc) - tterm[0, 0], 0.0)


def kernel(batch_feat, abs_proxy_label, camid, pseudo_cluster_label, memory,
           epoch, k, inter_loss_epoch):
    camid = camid.astype(jnp.int32)
    mabs = _target_rows(memory, abs_proxy_label.astype(jnp.int32))

    cams = jnp.arange(N_CAMS, dtype=jnp.int32)
    cnt = jnp.sum((camid[None, :] == cams[:, None]).astype(jnp.int32), axis=1)
    padded = ((cnt + TILE - 1) // TILE) * TILE
    ends = jnp.sum(jnp.where(cams[None, :] <= cams[:, None], padded[None, :], 0),
                   axis=1)
    starts = ends - padded
    tile_start = jnp.arange(NT, dtype=jnp.int32) * TILE
    tile_cam = jnp.minimum(
        jnp.sum((tile_start[:, None] >= ends[None, :]).astype(jnp.int32),
                axis=1), N_CAMS - 1)
    sel = (tile_cam[:, None] == cams[None, :])
    cnt_t = jnp.sum(jnp.where(sel, cnt[None, :], 0), axis=1)
    start_t = jnp.sum(jnp.where(sel, starts[None, :], 0), axis=1)
    vb_t = jnp.clip(start_t + cnt_t - tile_start, 0, TILE)
    n_real = ends[N_CAMS - 1] // TILE
    scalars = jnp.concatenate(
        [tile_cam, n_real[None], vb_t, jnp.maximum(cnt_t, 1)]).astype(jnp.int32)

    out = pl.pallas_call(
        _tile_kernel,
        grid_spec=pltpu.PrefetchScalarGridSpec(
            num_scalar_prefetch=1,
            grid=(1,),
            in_specs=[
                pl.BlockSpec((B, D), lambda i, tc: (0, 0)),
                pl.BlockSpec((N_PROXIES, D), lambda i, tc: (0, 0)),
                pl.BlockSpec((1, B), lambda i, tc: (0, 0)),
                pl.BlockSpec((B, D), lambda i, tc: (0, 0)),
            ],
            out_specs=pl.BlockSpec((1, 128), lambda i, tc: (0, 0)),
        ),
        out_shape=jax.ShapeDtypeStruct((1, 128), jnp.float32),
    )(scalars, batch_feat, memory, camid.reshape(1, B), mabs)
    return out[0, 0]


# speedup vs baseline: 1.9964x; 1.9948x over previous
import jax
import jax.numpy as jnp
from jax.experimental import pallas as pl
from jax.experimental.pallas import tpu as pltpu

N_PROXIES = 8192
N_CAMS = 8
PPC = N_PROXIES // N_CAMS
TEMP = 0.07
B = 1024
D = 256
TILE = 256
P = 3072
NT = P // TILE


def _tile_kernel(scal_ref, feat_ref, mem_ref, cam_ref, tgt_ref, out_ref):
    feat = feat_ref[...]
    camv = cam_ref[...]
    camsub = jax.lax.broadcasted_iota(jnp.int32, (N_CAMS, B), 0)
    ohi = (camsub == camv).astype(jnp.float32)
    cnt = jnp.sum(ohi, axis=1, keepdims=True)
    padded = jnp.floor((cnt + (TILE - 1)) * (1.0 / TILE)) * TILE
    r8 = jax.lax.broadcasted_iota(jnp.int32, (N_CAMS, N_CAMS), 0)
    c8 = jax.lax.broadcasted_iota(jnp.int32, (N_CAMS, N_CAMS), 1)
    strict_lt = (c8 < r8).astype(jnp.float32)
    starts = jax.lax.dot_general(
        strict_lt, padded, (((1,), (0,)), ((), ())),
        preferred_element_type=jnp.float32)
    ri = jax.lax.broadcasted_iota(jnp.int32, (B, B), 0)
    ci = jax.lax.broadcasted_iota(jnp.int32, (B, B), 1)
    lt_inc = (ri <= ci).astype(jnp.float32)
    incl = jax.lax.dot_general(
        ohi, lt_inc, (((1,), (0,)), ((), ())),
        preferred_element_type=jnp.float32)
    rank = jnp.sum(ohi * (incl - 1.0), axis=0, keepdims=True)
    pos = jnp.sum(ohi * starts, axis=0, keepdims=True) + rank
    winv = jnp.sum(jnp.where(ohi > 0, 1.0 / cnt, 0.0), axis=0,
                   keepdims=True)
    tgtf = (tgt_ref[...] - camv * PPC).astype(jnp.float32)

    def body(t, acc):
        c = scal_ref[t]
        pj = (jax.lax.broadcasted_iota(jnp.int32, (TILE, 1), 0)
              + t * TILE).astype(jnp.float32)
        gb = pos == pj
        g = gb.astype(jnp.float32)
        x = jax.lax.dot_general(
            g, feat, (((1,), (0,)), ((), ())),
            preferred_element_type=jnp.float32)
        tgt_t = jnp.sum(jnp.where(gb, tgtf, 0.0), axis=1, keepdims=True)
        w_t = jnp.sum(jnp.where(gb, winv, 0.0), axis=1, keepdims=True)
        w = mem_ref[pl.ds(c * PPC, PPC), :]
        sim = jax.lax.dot_general(
            x, w, (((1,), (1,)), ((), ())), preferred_element_type=jnp.float32
        ) * (1.0 / TEMP)
        lse = jnp.log(jnp.sum(jnp.exp(sim), axis=1, keepdims=True))
        cols = jax.lax.broadcasted_iota(jnp.int32, (TILE, PPC), 1)
        tlogit = jnp.sum(jnp.where(cols == tgt_t.astype(jnp.int32), sim, 0.0),
                         axis=1, keepdims=True)
        return acc + (lse - tlogit) * w_t

    n_real = scal_ref[NT]
    acc = jax.lax.fori_loop(0, n_real, body, jnp.zeros((TILE, 1), jnp.float32))
    lane = jax.lax.broadcasted_iota(jnp.int32, (1, 128), 1)
    out_ref[...] = jnp.where(lane == 0, jnp.sum(acc), 0.0)


def kernel(batch_feat, abs_proxy_label, camid, pseudo_cluster_label, memory,
           epoch, k, inter_loss_epoch):
    camid = camid.astype(jnp.int32)

    cams = jnp.arange(N_CAMS, dtype=jnp.int32)
    cnt = jnp.sum((camid[None, :] == cams[:, None]).astype(jnp.int32), axis=1)
    padded = ((cnt + TILE - 1) // TILE) * TILE
    ends = jnp.sum(jnp.where(cams[None, :] <= cams[:, None], padded[None, :], 0),
                   axis=1)
    tile_start = jnp.arange(NT, dtype=jnp.int32) * TILE
    tile_cam = jnp.minimum(
        jnp.sum((tile_start[:, None] >= ends[None, :]).astype(jnp.int32),
                axis=1), N_CAMS - 1)
    n_real = ends[N_CAMS - 1] // TILE
    scalars = jnp.concatenate([tile_cam, n_real[None]]).astype(jnp.int32)

    out = pl.pallas_call(
        _tile_kernel,
        grid_spec=pltpu.PrefetchScalarGridSpec(
            num_scalar_prefetch=1,
            grid=(1,),
            in_specs=[
                pl.BlockSpec((B, D), lambda i, tc: (0, 0)),
                pl.BlockSpec((N_PROXIES, D), lambda i, tc: (0, 0)),
                pl.BlockSpec((1, B), lambda i, tc: (0, 0)),
                pl.BlockSpec((1, B), lambda i, tc: (0, 0)),
            ],
            out_specs=pl.BlockSpec((1, 128), lambda i, tc: (0, 0)),
        ),
        out_shape=jax.ShapeDtypeStruct((1, 128), jnp.float32),
    )(scalars, batch_feat, memory,
      camid.reshape(1, B), abs_proxy_label.astype(jnp.int32).reshape(1, B))
    return out[0, 0]


# speedup vs baseline: 2.0077x; 1.0057x over previous
import jax
import jax.numpy as jnp
from jax.experimental import pallas as pl
from jax.experimental.pallas import tpu as pltpu

N_PROXIES = 8192
N_CAMS = 8
PPC = N_PROXIES // N_CAMS
TEMP = 0.07
B = 1024
D = 256
TILE = 256
P = 3072
NT = P // TILE


def _tile_kernel(scal_ref, feat_ref, mem_ref, cam_ref, tgt_ref, out_ref,
                 pos_s, winv_s, tgtf_s):
    c = pl.program_id(0)
    feat = feat_ref[...]

    @pl.when(c == 0)
    def _setup():
        camv = cam_ref[...]
        camsub = jax.lax.broadcasted_iota(jnp.int32, (N_CAMS, B), 0)
        ohi = (camsub == camv).astype(jnp.float32)
        cnt = jnp.sum(ohi, axis=1, keepdims=True)
        padded = jnp.floor((cnt + (TILE - 1)) * (1.0 / TILE)) * TILE
        r8 = jax.lax.broadcasted_iota(jnp.int32, (N_CAMS, N_CAMS), 0)
        c8 = jax.lax.broadcasted_iota(jnp.int32, (N_CAMS, N_CAMS), 1)
        strict_lt = (c8 < r8).astype(jnp.float32)
        starts = jax.lax.dot_general(
            strict_lt, padded, (((1,), (0,)), ((), ())),
            preferred_element_type=jnp.float32)
        ri = jax.lax.broadcasted_iota(jnp.int32, (B, B), 0)
        ci = jax.lax.broadcasted_iota(jnp.int32, (B, B), 1)
        lt_inc = (ri <= ci).astype(jnp.float32)
        incl = jax.lax.dot_general(
            ohi, lt_inc, (((1,), (0,)), ((), ())),
            preferred_element_type=jnp.float32)
        rank = jnp.sum(ohi * (incl - 1.0), axis=0, keepdims=True)
        pos_s[...] = jnp.sum(ohi * starts, axis=0, keepdims=True) + rank
        winv_s[...] = jnp.sum(jnp.where(ohi > 0, 1.0 / cnt, 0.0), axis=0,
                              keepdims=True)
        tgtf_s[...] = (tgt_ref[...] - camv * PPC).astype(jnp.float32)
        out_ref[...] = jnp.zeros_like(out_ref)

    pos = pos_s[...]
    winv = winv_s[...]
    tgtf = tgtf_s[...]
    w = mem_ref[...]

    def body(t, acc):
        pj = (jax.lax.broadcasted_iota(jnp.int32, (TILE, 1), 0)
              + t * TILE).astype(jnp.float32)
        gb = pos == pj
        g = gb.astype(jnp.float32)
        x = jax.lax.dot_general(
            g, feat, (((1,), (0,)), ((), ())),
            preferred_element_type=jnp.float32)
        tgt_t = jnp.sum(jnp.where(gb, tgtf, 0.0), axis=1, keepdims=True)
        w_t = jnp.sum(jnp.where(gb, winv, 0.0), axis=1, keepdims=True)
        sim = jax.lax.dot_general(
            x, w, (((1,), (1,)), ((), ())), preferred_element_type=jnp.float32
        ) * (1.0 / TEMP)
        lse = jnp.log(jnp.sum(jnp.exp(sim), axis=1, keepdims=True))
        cols = jax.lax.broadcasted_iota(jnp.int32, (TILE, PPC), 1)
        tlogit = jnp.sum(jnp.where(cols == tgt_t.astype(jnp.int32), sim, 0.0),
                         axis=1, keepdims=True)
        return acc + (lse - tlogit) * w_t

    acc = jax.lax.fori_loop(scal_ref[c], scal_ref[c + 1], body,
                            jnp.zeros((TILE, 1), jnp.float32))
    lane = jax.lax.broadcasted_iota(jnp.int32, (1, 128), 1)
    out_ref[...] += jnp.where(lane == 0, jnp.sum(acc), 0.0)


def kernel(batch_feat, abs_proxy_label, camid, pseudo_cluster_label, memory,
           epoch, k, inter_loss_epoch):
    camid = camid.astype(jnp.int32)

    cams = jnp.arange(N_CAMS, dtype=jnp.int32)
    cnt = jnp.sum((camid[None, :] == cams[:, None]).astype(jnp.int32), axis=1)
    ntiles = (cnt + TILE - 1) // TILE
    tb = jnp.sum(jnp.where(cams[None, :] < cams[:, None], ntiles[None, :], 0),
                 axis=1)
    scalars = jnp.concatenate([tb, (tb[N_CAMS - 1] + ntiles[N_CAMS - 1])[None]]
                              ).astype(jnp.int32)

    out = pl.pallas_call(
        _tile_kernel,
        grid_spec=pltpu.PrefetchScalarGridSpec(
            num_scalar_prefetch=1,
            grid=(N_CAMS,),
            in_specs=[
                pl.BlockSpec((B, D), lambda c, tc: (0, 0)),
                pl.BlockSpec((PPC, D), lambda c, tc: (c, 0)),
                pl.BlockSpec((1, B), lambda c, tc: (0, 0)),
                pl.BlockSpec((1, B), lambda c, tc: (0, 0)),
            ],
            out_specs=pl.BlockSpec((1, 128), lambda c, tc: (0, 0)),
            scratch_shapes=[
                pltpu.VMEM((1, B), jnp.float32),
                pltpu.VMEM((1, B), jnp.float32),
                pltpu.VMEM((1, B), jnp.float32),
            ],
        ),
        out_shape=jax.ShapeDtypeStruct((1, 128), jnp.float32),
    )(scalars, batch_feat, memory,
      camid.reshape(1, B), abs_proxy_label.astype(jnp.int32).reshape(1, B))
    return out[0, 0]
